# baseline (speedup 1.0000x reference)
import jax
import jax.numpy as jnp
from jax import lax
from jax.experimental import pallas as pl
from jax.experimental.pallas import tpu as pltpu
from jax.experimental.pallas import tpu_sc as plsc

C = 256
N = 160000
S = 1024

BN = 6400
NB = N // BN

NC = 2
NS = 16
NW = NC * NS
CPW = C // NW
CHUNK = 3200
NCHUNKS = N // CHUNK
VPC = CHUNK // 16
L = 16


W = 256


def _means_body(sp_ref, x_ref, out_ref, acc, cnt):
    j = pl.program_id(0)

    @pl.when(j == 0)
    def _init():
        acc[...] = jnp.zeros_like(acc)
        cnt[...] = jnp.zeros_like(cnt)

    sp_b = sp_ref[0, 0, :]
    xb = x_ref[...]
    smin = jnp.min(sp_b)
    smax = jnp.max(sp_b)
    off = pl.multiple_of(jnp.minimum((smin // 128) * 128, S - W), 128)
    in_window = (smax - off) < W

    @pl.when(in_window)
    def _narrow():
        rel = sp_b - off
        oh = (rel[:, None] == lax.broadcasted_iota(jnp.int32, (BN, W), 1)).astype(
            jnp.float32
        )
        acc[:, pl.ds(off, W)] += lax.dot_general(
            xb, oh, (((1,), (0,)), ((), ())), preferred_element_type=jnp.float32
        )
        cnt[:, pl.ds(off, W)] += jnp.sum(oh, axis=0, dtype=jnp.float32, keepdims=True)

    @pl.when(jnp.logical_not(in_window))
    def _full():
        for wo in range(0, S, W):
            oh = (
                (sp_b[:, None] - wo)
                == lax.broadcasted_iota(jnp.int32, (BN, W), 1)
            ).astype(jnp.float32)
            acc[:, wo : wo + W] += lax.dot_general(
                xb, oh, (((1,), (0,)), ((), ())), preferred_element_type=jnp.float32
            )
            cnt[:, wo : wo + W] += jnp.sum(
                oh, axis=0, dtype=jnp.float32, keepdims=True
            )

    @pl.when(j == NB - 1)
    def _emit():
        out_ref[...] = acc[...] * (1.0 / jnp.maximum(cnt[...], 1.0))


def _tc_means(x, sp3):
    return pl.pallas_call(
        _means_body,
        grid=(NB,),
        in_specs=[
            pl.BlockSpec((1, 1, BN), lambda j: (j, 0, 0)),
            pl.BlockSpec((C, BN), lambda j: (0, j)),
        ],
        out_specs=pl.BlockSpec((C, S), lambda j: (0, 0)),
        out_shape=jax.ShapeDtypeStruct((C, S), jnp.float32),
        scratch_shapes=[
            pltpu.VMEM((C, S), jnp.float32),
            pltpu.VMEM((1, S), jnp.float32),
        ],
    )(sp3, x)


def _bcast_body(means_hbm, sp_hbm, out_hbm, mbufs, obufs, sp_vs, msem, sp_sems, out_sems):
    wid = lax.axis_index("s") * NC + lax.axis_index("c")
    c0 = wid * CPW

    dms = [
        pltpu.async_copy(means_hbm.at[pl.ds((c0 + ci) * S, S)], mbufs[ci], msem)
        for ci in range(CPW)
    ]
    for d in dms:
        d.wait()

    for b in range(2):
        pltpu.async_copy(sp_hbm.at[pl.ds(b * CHUNK, CHUNK)], sp_vs[b], sp_sems[b])

    def pair_body(m, carry):
        for b in range(2):
            n0 = (2 * m + b) * CHUNK
            pltpu.make_async_copy(
                sp_hbm.at[pl.ds(n0, CHUNK)], sp_vs[b], sp_sems[b]
            ).wait()

            @pl.when(m > 0)
            def _drain(b=b, n0=n0):
                pltpu.make_async_copy(
                    obufs[b],
                    out_hbm.at[pl.ds(c0, CPW), pl.ds(n0 - 2 * CHUNK, CHUNK)],
                    out_sems[b],
                ).wait()

            @plsc.parallel_loop(0, VPC, unroll=8)
            def vec_body(j, b=b):
                off = j * L
                iv = sp_vs[b][pl.ds(off, L)]
                gs = [plsc.load_gather(mbufs[ci], [iv]) for ci in range(CPW)]
                for ci in range(CPW):
                    obufs[b][ci, pl.ds(off, L)] = gs[ci]

            pltpu.async_copy(
                obufs[b], out_hbm.at[pl.ds(c0, CPW), pl.ds(n0, CHUNK)], out_sems[b]
            )

            @pl.when(2 * m + b + 2 < NCHUNKS)
            def _prefetch(b=b, n0=n0):
                pltpu.async_copy(
                    sp_hbm.at[pl.ds(n0 + 2 * CHUNK, CHUNK)], sp_vs[b], sp_sems[b]
                )

        return carry

    lax.fori_loop(0, NCHUNKS // 2, pair_body, 0)

    for b in range(2):
        pltpu.make_async_copy(
            obufs[b],
            out_hbm.at[pl.ds(c0, CPW), pl.ds((NCHUNKS - 2 + b) * CHUNK, CHUNK)],
            out_sems[b],
        ).wait()


def _sc_broadcast(means_flat, sp):
    mesh = plsc.VectorSubcoreMesh(core_axis_name="c", subcore_axis_name="s")
    f = pl.kernel(
        _bcast_body,
        out_type=jax.ShapeDtypeStruct((C, N), jnp.float32),
        mesh=mesh,
        compiler_params=pltpu.CompilerParams(needs_layout_passes=False),
        scratch_types=[
            [pltpu.VMEM((S,), jnp.float32) for _ in range(CPW)],
            [pltpu.VMEM((CPW, CHUNK), jnp.float32) for _ in range(2)],
            [pltpu.VMEM((CHUNK,), jnp.int32) for _ in range(2)],
            pltpu.SemaphoreType.DMA,
            [pltpu.SemaphoreType.DMA for _ in range(2)],
            [pltpu.SemaphoreType.DMA for _ in range(2)],
        ],
    )
    return f(means_flat, sp)


@jax.jit
def _sp_norm(x, sp):
    means = _tc_means(x, sp.reshape(NB, 1, BN))
    return _sc_broadcast(means.reshape(C * S), sp)


def kernel(x, sp):
    return _sp_norm(x, sp)
